# baseline (speedup 1.0000x reference)
import functools

import jax
import jax.numpy as jnp
from jax import lax
from jax.experimental import pallas as pl
from jax.experimental.pallas import tpu as pltpu
from jax.experimental.pallas import tpu_sc as plsc

L = 16
E = 16
NS = 16


def _split_body(u_ref, i_ref, *outs_and_scratch):
    outs = outs_and_scratch[:2 * E]
    buf, sem_in, sem_out = outs_and_scratch[2 * E:]
    srcs = [(u_ref, 0, 0), (u_ref, 1, 0), (i_ref, 0, E), (i_ref, 1, E)]

    def in_copy(k):
        ref, strip, _ = srcs[k]
        return pltpu.make_async_copy(
            ref.at[pl.ds(8 * strip, 8)], buf.at[k % 2], sem_in)

    def out_copies(k):
        _, strip, off = srcs[k]
        return [pltpu.make_async_copy(
                    buf.at[k % 2].at[es], outs[off + 8 * strip + es], sem_out)
                for es in range(8)]

    pending_in = in_copy(0)
    pending_in.start()
    pending_out = [None, None]
    for k in range(4):
        if k >= 2:
            for c in pending_out[k % 2]:
                c.wait()
        pending_in.wait()
        if k < 3:
            pending_in = in_copy(k + 1)
            pending_in.start()
        ocs = out_copies(k)
        for c in ocs:
            c.start()
        pending_out[k % 2] = ocs
    for k in (2, 3):
        for c in pending_out[k % 2]:
            c.wait()


def _split_tables(u_t, i_t):
    n = u_t.shape[1]
    out = jax.ShapeDtypeStruct((n,), jnp.float32)
    return pl.pallas_call(
        _split_body,
        in_specs=[pl.BlockSpec(memory_space=pltpu.MemorySpace.HBM)] * 2,
        out_specs=[pl.BlockSpec(memory_space=pltpu.MemorySpace.HBM)] * (2 * E),
        out_shape=[out] * (2 * E),
        scratch_shapes=[pltpu.VMEM((2, 8, n), jnp.float32),
                        pltpu.SemaphoreType.DMA,
                        pltpu.SemaphoreType.DMA],
        compiler_params=pltpu.CompilerParams(
            vmem_limit_bytes=66 * 1024 * 1024),
    )(u_t, i_t)


def _sc_phase1():
    B = 16384
    NW = 32
    R = B // NW

    mesh = plsc.VectorSubcoreMesh(core_axis_name="c", subcore_axis_name="s",
                                  num_cores=2)

    @functools.partial(
        pl.kernel,
        out_type=[
            jax.ShapeDtypeStruct((NW, L), jnp.float32),
            jax.ShapeDtypeStruct((B,), jnp.float32),
        ],
        mesh=mesh,
        compiler_params=pltpu.CompilerParams(use_tc_tiling_on_sc=False),
        scratch_types=[
            pltpu.VMEM((R,), jnp.int32),
            pltpu.VMEM((R,), jnp.int32),
            pltpu.VMEM((E, R), jnp.float32),
            pltpu.VMEM((E, R), jnp.float32),
            pltpu.VMEM((R,), jnp.float32),
            pltpu.VMEM((R,), jnp.float32),
            pltpu.VMEM((R,), jnp.float32),
            pltpu.VMEM((L,), jnp.float32),
            pltpu.SemaphoreType.DMA,
        ],
    )
    def body(uid_h, iid_h, *rest):
        ucols = rest[:E]
        icols = rest[E:2 * E]
        (ub_h, ib_h, part_h, x_h,
         uid_v, iid_v, du, di, ubv, ibv, xv, accv, sem) = rest[2 * E:]
        wid = lax.axis_index("s") * 2 + lax.axis_index("c")
        base = wid * R

        pltpu.sync_copy(uid_h.at[pl.ds(base, R)], uid_v)
        pltpu.sync_copy(iid_h.at[pl.ds(base, R)], iid_v)

        copies = []
        for e in range(E):
            copies.append(pltpu.async_copy(ucols[e].at[uid_v], du.at[e], sem))
            copies.append(pltpu.async_copy(icols[e].at[iid_v], di.at[e], sem))
        copies.append(pltpu.async_copy(ub_h.at[uid_v], ubv, sem))
        copies.append(pltpu.async_copy(ib_h.at[iid_v], ibv, sem))
        for cp in copies:
            cp.wait()

        def dot_e(e):
            def dot_g(g, acc):
                return acc + (du[e, pl.ds(g * L, L)]
                              * di[e, pl.ds(g * L, L)])
            return lax.fori_loop(0, R // L, dot_g,
                                 jnp.zeros((L,), jnp.float32))

        acc = dot_e(0)
        for e in range(1, E):
            acc = acc + dot_e(e)
        accv[...] = acc
        pltpu.sync_copy(accv, part_h.at[wid])

        def bias_g(k, _):
            xv[pl.ds(k * L, L)] = (ubv[pl.ds(k * L, L)]
                                   + ibv[pl.ds(k * L, L)])
            return 0

        lax.fori_loop(0, R // L, bias_g, 0)
        pltpu.sync_copy(xv, x_h.at[pl.ds(base, R)])

    return body


def _sc_phase2():
    B = 16384
    NW = 32
    R = B // NS

    mesh = plsc.VectorSubcoreMesh(core_axis_name="c", subcore_axis_name="s",
                                  num_cores=1)

    @functools.partial(
        pl.kernel,
        out_type=jax.ShapeDtypeStruct((B,), jnp.float32),
        mesh=mesh,
        compiler_params=pltpu.CompilerParams(use_tc_tiling_on_sc=False),
        scratch_types=[
            pltpu.VMEM((NW, L), jnp.float32),
            pltpu.VMEM((R,), jnp.float32),
            pltpu.VMEM((R,), jnp.float32),
        ],
    )
    def body(part_h, x_h, out_h, allp, xv, outv):
        sid = lax.axis_index("s")
        base = sid * R

        pltpu.sync_copy(part_h, allp)
        pltpu.sync_copy(x_h.at[pl.ds(base, R)], xv)

        tot = allp[0]
        for j in range(1, NW):
            tot = tot + allp[j]
        lanes = lax.iota(jnp.int32, L)
        for shift in (1, 2, 4, 8):
            tot = tot + tot.at[(lanes + shift) % L].get(
                mode="promise_in_bounds")
        s = tot

        def out_g(k, _):
            x = s + xv[pl.ds(k * L, L)]
            outv[pl.ds(k * L, L)] = 1.0 / (1.0 + jnp.exp(-x))
            return 0

        lax.fori_loop(0, R // L, out_g, 0)
        pltpu.sync_copy(outv, out_h.at[pl.ds(base, R)])

    return body


def kernel(inputs, user_embedding, user_bias, item_embedding, item_bias):
    B = inputs.shape[0]
    ii = inputs.astype(jnp.int32)
    uid = ii[:, 0]
    iid = ii[:, 1]
    cols = _split_tables(user_embedding.T, item_embedding.T)
    ucols, icols = cols[:E], cols[E:]
    ub = user_bias.reshape(-1)
    ib = item_bias.reshape(-1)
    part, x = _sc_phase1()(uid, iid, *ucols, *icols, ub, ib)
    out = _sc_phase2()(part, x)
    return out.reshape(B, 1)


# speedup vs baseline: 1.0188x; 1.0188x over previous
import functools

import jax
import jax.numpy as jnp
from jax import lax
from jax.experimental import pallas as pl
from jax.experimental.pallas import tpu as pltpu
from jax.experimental.pallas import tpu_sc as plsc

L = 16
E = 16
NS = 16


def _split_body(in_ref, *outs_and_sem):
    outs = outs_and_sem[:E]
    sem = outs_and_sem[E]
    h = pl.program_id(0)
    for hh in range(2):
        @pl.when(h == hh)
        def _():
            cps = [pltpu.make_async_copy(in_ref.at[es], outs[hh * 8 + es],
                                         sem)
                   for es in range(8)]
            for c in cps:
                c.start()
            for c in cps:
                c.wait()


def _split_one(table_t):
    n = table_t.shape[1]
    out = jax.ShapeDtypeStruct((n,), jnp.float32)
    return pl.pallas_call(
        _split_body,
        grid=(2,),
        in_specs=[pl.BlockSpec((8, n), lambda h: (h, 0))],
        out_specs=[pl.BlockSpec(memory_space=pltpu.MemorySpace.HBM)] * E,
        out_shape=[out] * E,
        scratch_shapes=[pltpu.SemaphoreType.DMA],
        compiler_params=pltpu.CompilerParams(
            vmem_limit_bytes=120 * 1024 * 1024),
    )(table_t)


def _sc_phase1():
    B = 16384
    NW = 32
    R = B // NW

    mesh = plsc.VectorSubcoreMesh(core_axis_name="c", subcore_axis_name="s",
                                  num_cores=2)

    @functools.partial(
        pl.kernel,
        out_type=[
            jax.ShapeDtypeStruct((NW, L), jnp.float32),
            jax.ShapeDtypeStruct((B,), jnp.float32),
        ],
        mesh=mesh,
        compiler_params=pltpu.CompilerParams(use_tc_tiling_on_sc=False),
        scratch_types=[
            pltpu.VMEM((R,), jnp.int32),
            pltpu.VMEM((R,), jnp.int32),
            pltpu.VMEM((E, R), jnp.float32),
            pltpu.VMEM((E, R), jnp.float32),
            pltpu.VMEM((R,), jnp.float32),
            pltpu.VMEM((R,), jnp.float32),
            pltpu.VMEM((R,), jnp.float32),
            pltpu.VMEM((L,), jnp.float32),
            pltpu.SemaphoreType.DMA,
        ],
    )
    def body(uid_h, iid_h, *rest):
        ucols = rest[:E]
        icols = rest[E:2 * E]
        (ub_h, ib_h, part_h, x_h,
         uid_v, iid_v, du, di, ubv, ibv, xv, accv, sem) = rest[2 * E:]
        wid = lax.axis_index("s") * 2 + lax.axis_index("c")
        base = wid * R

        pltpu.sync_copy(uid_h.at[pl.ds(base, R)], uid_v)
        pltpu.sync_copy(iid_h.at[pl.ds(base, R)], iid_v)

        copies = []
        for e in range(E):
            copies.append(pltpu.async_copy(ucols[e].at[uid_v], du.at[e], sem))
            copies.append(pltpu.async_copy(icols[e].at[iid_v], di.at[e], sem))
        copies.append(pltpu.async_copy(ub_h.at[uid_v], ubv, sem))
        copies.append(pltpu.async_copy(ib_h.at[iid_v], ibv, sem))
        for cp in copies:
            cp.wait()

        def dot_e(e):
            def dot_g(g, acc):
                return acc + (du[e, pl.ds(g * L, L)]
                              * di[e, pl.ds(g * L, L)])
            return lax.fori_loop(0, R // L, dot_g,
                                 jnp.zeros((L,), jnp.float32))

        acc = dot_e(0)
        for e in range(1, E):
            acc = acc + dot_e(e)
        accv[...] = acc
        pltpu.sync_copy(accv, part_h.at[wid])

        def bias_g(k, _):
            xv[pl.ds(k * L, L)] = (ubv[pl.ds(k * L, L)]
                                   + ibv[pl.ds(k * L, L)])
            return 0

        lax.fori_loop(0, R // L, bias_g, 0)
        pltpu.sync_copy(xv, x_h.at[pl.ds(base, R)])

    return body


def _sc_phase2():
    B = 16384
    NW = 32
    R = B // NS

    mesh = plsc.VectorSubcoreMesh(core_axis_name="c", subcore_axis_name="s",
                                  num_cores=1)

    @functools.partial(
        pl.kernel,
        out_type=jax.ShapeDtypeStruct((B,), jnp.float32),
        mesh=mesh,
        compiler_params=pltpu.CompilerParams(use_tc_tiling_on_sc=False),
        scratch_types=[
            pltpu.VMEM((NW, L), jnp.float32),
            pltpu.VMEM((R,), jnp.float32),
            pltpu.VMEM((R,), jnp.float32),
        ],
    )
    def body(part_h, x_h, out_h, allp, xv, outv):
        sid = lax.axis_index("s")
        base = sid * R

        pltpu.sync_copy(part_h, allp)
        pltpu.sync_copy(x_h.at[pl.ds(base, R)], xv)

        tot = allp[0]
        for j in range(1, NW):
            tot = tot + allp[j]
        lanes = lax.iota(jnp.int32, L)
        for shift in (1, 2, 4, 8):
            tot = tot + tot.at[(lanes + shift) % L].get(
                mode="promise_in_bounds")
        s = tot

        def out_g(k, _):
            x = s + xv[pl.ds(k * L, L)]
            outv[pl.ds(k * L, L)] = 1.0 / (1.0 + jnp.exp(-x))
            return 0

        lax.fori_loop(0, R // L, out_g, 0)
        pltpu.sync_copy(outv, out_h.at[pl.ds(base, R)])

    return body


def kernel(inputs, user_embedding, user_bias, item_embedding, item_bias):
    B = inputs.shape[0]
    ii = inputs.astype(jnp.int32)
    uid = ii[:, 0]
    iid = ii[:, 1]
    ucols = _split_one(user_embedding.T)
    icols = _split_one(item_embedding.T)
    ub = user_bias.reshape(-1)
    ib = item_bias.reshape(-1)
    part, x = _sc_phase1()(uid, iid, *ucols, *icols, ub, ib)
    out = _sc_phase2()(part, x)
    return out.reshape(B, 1)
